# baseline (speedup 1.0000x reference)
import functools

import jax
import jax.numpy as jnp
from jax import lax
from jax.experimental import pallas as pl
from jax.experimental.pallas import tpu as pltpu
from jax.experimental.pallas import tpu_sc as plsc

P, C, D = 64, 64, 256
S, K = 2048, 2
NC, NS, L = 2, 16, 16
TPW = S // NS
DH = D // NC
PPW = P // NS
E = K * TPW

_DIMNUMS = lax.GatherDimensionNumbers(
    offset_dims=(), collapsed_slice_dims=(0,), start_index_map=(0,))


def _permute(v, idx):
    return lax.gather(v, idx[:, None], _DIMNUMS, (1,),
                      mode=lax.GatherScatterMode.PROMISE_IN_BOUNDS)


def _tc_dense_body(keys_ref, states_ref, ksum_ref, smean_ref):
    ksum_ref[...] = jnp.sum(keys_ref[...], axis=1) * (1.0 / C)
    smean_ref[...] = jnp.mean(states_ref[...], axis=1)


def _tc_dense(keys2, states):
    return pl.pallas_call(
        _tc_dense_body,
        out_shape=(
            jax.ShapeDtypeStruct((S * K,), jnp.float32),
            jax.ShapeDtypeStruct((P, D), jnp.float32),
        ),
    )(keys2, states)


_sc_mesh = plsc.VectorSubcoreMesh(core_axis_name="c", subcore_axis_name="s")


@functools.partial(
    pl.kernel,
    mesh=_sc_mesh,
    out_type=jax.ShapeDtypeStruct((S, D), jnp.float32),
    scratch_types=[
        pltpu.VMEM((E, DH), jnp.float32),
        pltpu.VMEM((TPW, DH), jnp.float32),
        pltpu.VMEM((E,), jnp.float32),
        pltpu.VMEM((E,), jnp.int32),
        pltpu.VMEM((PPW, DH), jnp.float32),
        pltpu.VMEM_SHARED((P, DH), jnp.float32),
    ],
)
def _sc_scatter_gather(idx_hbm, ksum_hbm, values_hbm, q_hbm, smean_hbm,
                       out_hbm, sbuf, vbuf, ksbuf, ibuf, tbuf, shared):
    cid = lax.axis_index("c")
    sid = lax.axis_index("s")
    t0 = sid * TPW
    c0 = cid * DH
    p0 = sid * PPW
    e0 = t0 * K

    pltpu.sync_copy(smean_hbm.at[pl.ds(p0, PPW), pl.ds(c0, DH)], tbuf)
    pltpu.sync_copy(tbuf, shared.at[pl.ds(p0, PPW)])
    pltpu.sync_copy(idx_hbm.at[pl.ds(e0, E)], ibuf)
    pltpu.sync_copy(ksum_hbm.at[pl.ds(e0, E)], ksbuf)
    pltpu.sync_copy(values_hbm.at[pl.ds(t0, TPW), pl.ds(c0, DH)], vbuf)
    plsc.subcore_barrier()

    bidx = tuple(jnp.full((L,), u, jnp.int32) for u in range(L))

    def grp(g, carry):
        kv = ksbuf[pl.ds(g * L, L)]
        for u in range(L):
            sv = _permute(kv, bidx[u])
            t = g * (L // K) + u // K
            for j in range(DH // L):
                sl = pl.ds(j * L, L)
                sbuf[g * L + u, sl] = sv * vbuf[t, sl]
        return carry

    lax.fori_loop(0, E // L, grp, 0)
    pltpu.sync_copy(sbuf, shared.at[ibuf], add=True)
    pltpu.sync_copy(q_hbm.at[pl.ds(t0, TPW), pl.ds(c0, DH)], vbuf)
    plsc.subcore_barrier()

    pltpu.sync_copy(shared.at[ibuf], sbuf)

    def tok(t, carry):
        for j in range(DH // L):
            sl = pl.ds(j * L, L)
            g = sbuf[t * K, sl] + sbuf[t * K + 1, sl]
            vbuf[t, sl] = g * vbuf[t, sl]
        return carry

    lax.fori_loop(0, TPW, tok, 0)
    pltpu.sync_copy(vbuf, out_hbm.at[pl.ds(t0, TPW), pl.ds(c0, DH)])


def kernel(partition_indices, keys, values, queries, states):
    b, s, k = partition_indices.shape
    assert (b, s, k) == (1, S, K)
    idx2 = partition_indices.reshape(S * K).astype(jnp.int32)
    keys2 = keys.reshape(S * K, C)
    values2 = values.reshape(S, D)
    queries2 = queries.reshape(S, D)

    ksum, smean = _tc_dense(keys2, states)
    out2 = _sc_scatter_gather(idx2, ksum, values2, queries2, smean)
    return out2.reshape(1, S, D)


# speedup vs baseline: 1.1524x; 1.1524x over previous
import functools

import jax
import jax.numpy as jnp
from jax import lax
from jax.experimental import pallas as pl
from jax.experimental.pallas import tpu as pltpu
from jax.experimental.pallas import tpu_sc as plsc

P, C, D = 64, 64, 256
S, K = 2048, 2
NC, NS, L = 2, 16, 16
TPW = S // NS
DH = D // NC
PPW = P // NS
E = K * TPW


def _tc_dense_body(keys_ref, values_ref, states_ref, scaled_ref, smean_ref):
    ks = jnp.sum(keys_ref[...], axis=1) * (1.0 / C)
    v = values_ref[...]
    scaled = ks.reshape(S, K)[:, :, None] * v[:, None, :]
    scaled_ref[...] = scaled.reshape(S * K, D)
    smean_ref[...] = jnp.mean(states_ref[...], axis=1)


def _tc_dense(keys2, values2, states):
    return pl.pallas_call(
        _tc_dense_body,
        out_shape=(
            jax.ShapeDtypeStruct((S * K, D), jnp.float32),
            jax.ShapeDtypeStruct((P, D), jnp.float32),
        ),
    )(keys2, values2, states)


_sc_mesh = plsc.VectorSubcoreMesh(core_axis_name="c", subcore_axis_name="s")


@functools.partial(
    pl.kernel,
    mesh=_sc_mesh,
    out_type=jax.ShapeDtypeStruct((S, D), jnp.float32),
    scratch_types=[
        pltpu.VMEM((E, DH), jnp.float32),
        pltpu.VMEM((TPW, DH), jnp.float32),
        pltpu.VMEM((E,), jnp.int32),
        pltpu.VMEM((PPW, DH), jnp.float32),
        pltpu.VMEM_SHARED((P, DH), jnp.float32),
    ],
)
def _sc_scatter_gather(idx_hbm, scaled_hbm, q_hbm, smean_hbm, out_hbm,
                       sbuf, vbuf, ibuf, tbuf, shared):
    cid = lax.axis_index("c")
    sid = lax.axis_index("s")
    t0 = sid * TPW
    c0 = cid * DH
    p0 = sid * PPW
    e0 = t0 * K

    pltpu.sync_copy(smean_hbm.at[pl.ds(p0, PPW), pl.ds(c0, DH)], tbuf)
    pltpu.sync_copy(tbuf, shared.at[pl.ds(p0, PPW)])
    pltpu.sync_copy(idx_hbm.at[pl.ds(e0, E)], ibuf)
    pltpu.sync_copy(scaled_hbm.at[pl.ds(e0, E), pl.ds(c0, DH)], sbuf)
    plsc.subcore_barrier()

    pltpu.sync_copy(sbuf, shared.at[ibuf], add=True)
    pltpu.sync_copy(q_hbm.at[pl.ds(t0, TPW), pl.ds(c0, DH)], vbuf)
    plsc.subcore_barrier()

    pltpu.sync_copy(shared.at[ibuf], sbuf)

    def tok(t, carry):
        for j in range(DH // L):
            sl = pl.ds(j * L, L)
            g = sbuf[t * K, sl] + sbuf[t * K + 1, sl]
            vbuf[t, sl] = g * vbuf[t, sl]
        return carry

    lax.fori_loop(0, TPW, tok, 0)
    pltpu.sync_copy(vbuf, out_hbm.at[pl.ds(t0, TPW), pl.ds(c0, DH)])


def kernel(partition_indices, keys, values, queries, states):
    b, s, k = partition_indices.shape
    assert (b, s, k) == (1, S, K)
    idx2 = partition_indices.reshape(S * K).astype(jnp.int32)
    keys2 = keys.reshape(S * K, C)
    values2 = values.reshape(S, D)
    queries2 = queries.reshape(S, D)

    scaled, smean = _tc_dense(keys2, values2, states)
    out2 = _sc_scatter_gather(idx2, scaled, queries2, smean)
    return out2.reshape(1, S, D)


# speedup vs baseline: 1.3880x; 1.2044x over previous
import functools

import jax
import jax.numpy as jnp
from jax import lax
from jax.experimental import pallas as pl
from jax.experimental.pallas import tpu as pltpu
from jax.experimental.pallas import tpu_sc as plsc

P, C, D = 64, 64, 256
S, K = 2048, 2
NC, NS, L = 2, 16, 16
TPW = S // NS
DH = D // NC
PPW = P // NS


def _tc_dense_body(keys0_ref, keys1_ref, values_ref, states_ref,
                   scaled_ref, smean_ref):
    v = values_ref[...]
    ks0 = jnp.sum(keys0_ref[...], axis=1, keepdims=True) * (1.0 / C)
    ks1 = jnp.sum(keys1_ref[...], axis=1, keepdims=True) * (1.0 / C)
    scaled_ref[0] = ks0 * v
    scaled_ref[1] = ks1 * v
    smean_ref[...] = jnp.mean(states_ref[...], axis=1)


def _tc_dense(keys, values, states):
    return pl.pallas_call(
        _tc_dense_body,
        out_shape=(
            jax.ShapeDtypeStruct((K, S, D), jnp.float32),
            jax.ShapeDtypeStruct((P, D), jnp.float32),
        ),
    )(keys[:, 0, :], keys[:, 1, :], values, states)


_sc_mesh = plsc.VectorSubcoreMesh(core_axis_name="c", subcore_axis_name="s")


@functools.partial(
    pl.kernel,
    mesh=_sc_mesh,
    out_type=jax.ShapeDtypeStruct((S, D), jnp.float32),
    scratch_types=[
        pltpu.VMEM((K * TPW, DH), jnp.float32),
        pltpu.VMEM((TPW, DH), jnp.float32),
        pltpu.VMEM((K, TPW), jnp.int32),
        pltpu.VMEM((PPW, DH), jnp.float32),
        pltpu.VMEM_SHARED((P, DH), jnp.float32),
    ],
)
def _sc_scatter_gather(idx_hbm, scaled_hbm, smean_hbm, q_hbm, out_hbm,
                       sbuf, qbuf, ibuf, tbuf, shared):
    cid = lax.axis_index("c")
    sid = lax.axis_index("s")
    t0 = sid * TPW
    c0 = cid * DH

    pltpu.sync_copy(smean_hbm.at[pl.ds(sid * PPW, PPW), pl.ds(c0, DH)], tbuf)
    pltpu.sync_copy(tbuf, shared.at[pl.ds(sid * PPW, PPW)])
    pltpu.sync_copy(idx_hbm.at[sid], ibuf)
    plsc.subcore_barrier()

    for k in range(K):
        pltpu.sync_copy(
            scaled_hbm.at[k, pl.ds(t0, TPW), pl.ds(c0, DH)],
            sbuf.at[pl.ds(k * TPW, TPW)],
        )
    for k in range(K):
        pltpu.sync_copy(sbuf.at[pl.ds(k * TPW, TPW)],
                        shared.at[ibuf.at[k]], add=True)
    plsc.subcore_barrier()

    for k in range(K):
        pltpu.sync_copy(shared.at[ibuf.at[k]],
                        sbuf.at[pl.ds(k * TPW, TPW)])
    pltpu.sync_copy(q_hbm.at[pl.ds(t0, TPW), pl.ds(c0, DH)], qbuf)

    def tok(t, carry):
        for j in range(DH // L):
            sl = pl.ds(j * L, L)
            g = sbuf[t, sl] + sbuf[TPW + t, sl]
            qbuf[t, sl] = g * qbuf[t, sl]
        return carry

    lax.fori_loop(0, TPW, tok, 0)
    pltpu.sync_copy(qbuf, out_hbm.at[pl.ds(t0, TPW), pl.ds(c0, DH)])


def kernel(partition_indices, keys, values, queries, states):
    b, s, k = partition_indices.shape
    assert (b, s, k) == (1, S, K)
    keys2 = keys.reshape(S, K, C)
    values2 = values.reshape(S, D)
    queries2 = queries.reshape(S, D)

    scaled, smean = _tc_dense(keys2, values2, states)

    idx3 = (partition_indices.reshape(S, K)
            .reshape(NS, TPW, K)
            .transpose(0, 2, 1)
            .astype(jnp.int32))

    out2 = _sc_scatter_gather(idx3, scaled, smean, queries2)
    return out2.reshape(1, S, D)


# speedup vs baseline: 1.5035x; 1.0832x over previous
import functools

import jax
import jax.numpy as jnp
from jax import lax
from jax.experimental import pallas as pl
from jax.experimental.pallas import tpu as pltpu
from jax.experimental.pallas import tpu_sc as plsc

P, C, D = 64, 64, 256
S, K = 2048, 2
NC, NS, L = 2, 16, 16
TPW = S // NS
DH = D // NC
PPW = P // NS


def _tc_dense_body(keys0_ref, keys1_ref, values_ref, states_ref,
                   scaled_ref, smean_ref):
    v = values_ref[...]
    ks0 = jnp.sum(keys0_ref[...], axis=1, keepdims=True) * (1.0 / C)
    ks1 = jnp.sum(keys1_ref[...], axis=1, keepdims=True) * (1.0 / C)
    scaled_ref[0] = ks0 * v
    scaled_ref[1] = ks1 * v
    smean_ref[...] = jnp.mean(states_ref[...], axis=1)


def _tc_dense(keys, values, states):
    return pl.pallas_call(
        _tc_dense_body,
        out_shape=(
            jax.ShapeDtypeStruct((K, S, D), jnp.float32),
            jax.ShapeDtypeStruct((P, D), jnp.float32),
        ),
    )(keys[:, 0, :], keys[:, 1, :], values, states)


_sc_mesh = plsc.VectorSubcoreMesh(core_axis_name="c", subcore_axis_name="s")


@functools.partial(
    pl.kernel,
    mesh=_sc_mesh,
    out_type=jax.ShapeDtypeStruct((S, D), jnp.float32),
    scratch_types=[
        pltpu.VMEM((K * TPW, DH), jnp.float32),
        pltpu.VMEM((TPW, DH), jnp.float32),
        pltpu.VMEM((K, TPW), jnp.int32),
        pltpu.VMEM((PPW, DH), jnp.float32),
        pltpu.VMEM_SHARED((P, DH), jnp.float32),
        pltpu.SemaphoreType.DMA,
        pltpu.SemaphoreType.DMA,
    ],
)
def _sc_scatter_gather(idx_hbm, scaled_hbm, smean_hbm, q_hbm, out_hbm,
                       sbuf, qbuf, ibuf, tbuf, shared, ssem, qsem):
    cid = lax.axis_index("c")
    sid = lax.axis_index("s")
    t0 = sid * TPW
    c0 = cid * DH

    scp = [
        pltpu.async_copy(
            scaled_hbm.at[k, pl.ds(t0, TPW), pl.ds(c0, DH)],
            sbuf.at[pl.ds(k * TPW, TPW)],
            ssem,
        )
        for k in range(K)
    ]
    qcp = pltpu.async_copy(q_hbm.at[pl.ds(t0, TPW), pl.ds(c0, DH)], qbuf, qsem)
    pltpu.sync_copy(smean_hbm.at[pl.ds(sid * PPW, PPW), pl.ds(c0, DH)], tbuf)
    pltpu.sync_copy(tbuf, shared.at[pl.ds(sid * PPW, PPW)])
    pltpu.sync_copy(idx_hbm.at[sid], ibuf)
    plsc.subcore_barrier()

    for cp in scp:
        cp.wait()
    for k in range(K):
        pltpu.sync_copy(sbuf.at[pl.ds(k * TPW, TPW)],
                        shared.at[ibuf.at[k]], add=True)
    plsc.subcore_barrier()

    for k in range(K):
        pltpu.sync_copy(shared.at[ibuf.at[k]],
                        sbuf.at[pl.ds(k * TPW, TPW)])
    qcp.wait()

    def tok(t, carry):
        for j in range(DH // L):
            sl = pl.ds(j * L, L)
            g = sbuf[t, sl] + sbuf[TPW + t, sl]
            qbuf[t, sl] = g * qbuf[t, sl]
        return carry

    lax.fori_loop(0, TPW, tok, 0)
    pltpu.sync_copy(qbuf, out_hbm.at[pl.ds(t0, TPW), pl.ds(c0, DH)])


def kernel(partition_indices, keys, values, queries, states):
    b, s, k = partition_indices.shape
    assert (b, s, k) == (1, S, K)
    keys2 = keys.reshape(S, K, C)
    values2 = values.reshape(S, D)
    queries2 = queries.reshape(S, D)

    scaled, smean = _tc_dense(keys2, values2, states)

    idx3 = (partition_indices.reshape(S, K)
            .reshape(NS, TPW, K)
            .transpose(0, 2, 1)
            .astype(jnp.int32))

    out2 = _sc_scatter_gather(idx3, scaled, smean, queries2)
    return out2.reshape(1, S, D)


# speedup vs baseline: 1.5042x; 1.0005x over previous
import functools

import jax
import jax.numpy as jnp
from jax import lax
from jax.experimental import pallas as pl
from jax.experimental.pallas import tpu as pltpu
from jax.experimental.pallas import tpu_sc as plsc

P, C, D = 64, 64, 256
S, K = 2048, 2
NC, NS, L = 2, 16, 16
TPW = S // NS
DH = D // NC
PPW = P // NS


def _tc_dense_body(keys0_ref, keys1_ref, values_ref, states_ref,
                   scaled_ref, smean_ref):
    v = values_ref[...]
    ks0 = jnp.sum(keys0_ref[...], axis=1, keepdims=True) * (1.0 / C)
    ks1 = jnp.sum(keys1_ref[...], axis=1, keepdims=True) * (1.0 / C)
    scaled_ref[0] = ks0 * v
    scaled_ref[1] = ks1 * v
    smean_ref[...] = jnp.mean(states_ref[...], axis=1)


def _tc_dense(keys, values, states):
    return pl.pallas_call(
        _tc_dense_body,
        out_shape=(
            jax.ShapeDtypeStruct((K, S, D), jnp.float32),
            jax.ShapeDtypeStruct((P, D), jnp.float32),
        ),
    )(keys[:, 0, :], keys[:, 1, :], values, states)


_sc_mesh = plsc.VectorSubcoreMesh(core_axis_name="c", subcore_axis_name="s")


@functools.partial(
    pl.kernel,
    mesh=_sc_mesh,
    out_type=jax.ShapeDtypeStruct((S, D), jnp.float32),
    scratch_types=[
        pltpu.VMEM((K * TPW, DH), jnp.float32),
        pltpu.VMEM((TPW, DH), jnp.float32),
        pltpu.VMEM((K, TPW), jnp.int32),
        pltpu.VMEM((PPW, DH), jnp.float32),
        pltpu.VMEM_SHARED((P, DH), jnp.float32),
        pltpu.SemaphoreType.DMA,
        pltpu.SemaphoreType.DMA,
        pltpu.SemaphoreType.DMA,
        pltpu.SemaphoreType.DMA,
    ],
)
def _sc_scatter_gather(idx_hbm, scaled_hbm, smean_hbm, q_hbm, out_hbm,
                       sbuf, qbuf, ibuf, tbuf, shared, ssem, qsem,
                       gsem0, gsem1):
    cid = lax.axis_index("c")
    sid = lax.axis_index("s")
    t0 = sid * TPW
    c0 = cid * DH

    scp = [
        pltpu.async_copy(
            scaled_hbm.at[k, pl.ds(t0, TPW), pl.ds(c0, DH)],
            sbuf.at[pl.ds(k * TPW, TPW)],
            ssem,
        )
        for k in range(K)
    ]
    qcp = pltpu.async_copy(q_hbm.at[pl.ds(t0, TPW), pl.ds(c0, DH)], qbuf, qsem)
    pltpu.sync_copy(smean_hbm.at[pl.ds(sid * PPW, PPW), pl.ds(c0, DH)], tbuf)
    pltpu.sync_copy(tbuf, shared.at[pl.ds(sid * PPW, PPW)])
    pltpu.sync_copy(idx_hbm.at[sid], ibuf)
    plsc.subcore_barrier()

    for cp in scp:
        cp.wait()
    for k in range(K):
        pltpu.sync_copy(sbuf.at[pl.ds(k * TPW, TPW)],
                        shared.at[ibuf.at[k]], add=True)
    plsc.subcore_barrier()

    H = TPW // 2
    gsems = (gsem0, gsem1)
    gcps = [
        [
            pltpu.async_copy(
                shared.at[ibuf.at[k, pl.ds(h * H, H)]],
                sbuf.at[pl.ds(k * TPW + h * H, H)],
                gsems[h],
            )
            for k in range(K)
        ]
        for h in range(2)
    ]
    qcp.wait()

    def tok(t, carry):
        for j in range(DH // L):
            sl = pl.ds(j * L, L)
            g = sbuf[t, sl] + sbuf[TPW + t, sl]
            qbuf[t, sl] = g * qbuf[t, sl]
        return carry

    for cp in gcps[0]:
        cp.wait()
    lax.fori_loop(0, H, tok, 0)
    for cp in gcps[1]:
        cp.wait()
    lax.fori_loop(H, TPW, tok, 0)
    pltpu.sync_copy(qbuf, out_hbm.at[pl.ds(t0, TPW), pl.ds(c0, DH)])


def kernel(partition_indices, keys, values, queries, states):
    b, s, k = partition_indices.shape
    assert (b, s, k) == (1, S, K)
    keys2 = keys.reshape(S, K, C)
    values2 = values.reshape(S, D)
    queries2 = queries.reshape(S, D)

    scaled, smean = _tc_dense(keys2, values2, states)

    idx3 = (partition_indices.reshape(S, K)
            .reshape(NS, TPW, K)
            .transpose(0, 2, 1)
            .astype(jnp.int32))

    out2 = _sc_scatter_gather(idx3, scaled, smean, queries2)
    return out2.reshape(1, S, D)
